# baseline (speedup 1.0000x reference)
import functools

import jax
import jax.numpy as jnp
from jax import lax
from jax.experimental import pallas as pl
from jax.experimental.pallas import tpu as pltpu
from jax.experimental.pallas import tpu_sc as plsc

N, E, D, G = 10000, 320000, 128, 256
NC, NS = 2, 16
NW = NC * NS
EPW = E // NW
CH = 80
NCHUNK = EPW // CH
NP = 10240
RPS = NP // NS
ZR = 128
NLANE = 16
NSL = D // NLANE


NBUF = 4
NMAIN = (NCHUNK // NBUF) * NBUF


def _sc_agg_body(x_hbm, src_hbm, dst_hbm, w_hbm, out0_hbm, out1_hbm,
                 agg_sh, rows, src_r, dst_r, w_r, gsems, ssems, isems):
    c = lax.axis_index("c")
    s = lax.axis_index("s")
    wid = c * NS + s

    base = wid * EPW

    def zrow(i, carry):
        for sl in range(NSL):
            rows[NBUF - 1][i, pl.ds(sl * NLANE, NLANE)] = jnp.zeros(
                (NLANE,), jnp.float32)
        return carry

    lax.fori_loop(0, CH, zrow, 0)
    for i in range(RPS // CH):
        pltpu.async_copy(rows[NBUF - 1],
                         agg_sh.at[pl.ds(s * RPS + i * CH, CH)], ssems[0])

    def issue_idx(k, r):
        e0 = base + k * CH
        pltpu.async_copy(src_hbm.at[pl.ds(e0, CH)], src_r.at[r], isems[r])
        pltpu.async_copy(dst_hbm.at[pl.ds(e0, CH)], dst_r.at[r], isems[r])
        pltpu.async_copy(w_hbm.at[pl.ds(e0, CH)], w_r.at[r], isems[r])

    def wait_idx(k, r):
        e0 = base + k * CH
        pltpu.make_async_copy(src_hbm.at[pl.ds(e0, CH)], src_r.at[r], isems[r]).wait()
        pltpu.make_async_copy(dst_hbm.at[pl.ds(e0, CH)], dst_r.at[r], isems[r]).wait()
        pltpu.make_async_copy(w_hbm.at[pl.ds(e0, CH)], w_r.at[r], isems[r]).wait()

    def issue_gather(r, b):
        pltpu.async_copy(x_hbm.at[src_r.at[r]], rows[b], gsems[b])

    def wait_gather(r, b):
        pltpu.make_async_copy(x_hbm.at[src_r.at[r]], rows[b], gsems[b]).wait()

    def issue_scatter(r, b):
        pltpu.async_copy(rows[b], agg_sh.at[dst_r.at[r]], ssems[b], add=True)

    def wait_scatter(r, b):
        pltpu.make_async_copy(rows[b], agg_sh.at[dst_r.at[r]], ssems[b]).wait()

    def scale_chunk(r, b):
        @plsc.parallel_loop(0, CH // NLANE, 1)
        def _(g):
            wvec = w_r[r, pl.ds(g * NLANE, NLANE)]
            for j in range(NLANE):
                e = g * NLANE + j
                w = wvec[j]
                for sl in range(NSL):
                    rows[b][e, pl.ds(sl * NLANE, NLANE)] = (
                        rows[b][e, pl.ds(sl * NLANE, NLANE)] * w)

    issue_idx(0, 0)
    issue_idx(1, 1)
    wait_idx(0, 0)
    issue_gather(0, 0)
    for i in range(RPS // CH):
        pltpu.make_async_copy(rows[NBUF - 1],
                              agg_sh.at[pl.ds(s * RPS + i * CH, CH)],
                              ssems[0]).wait()
    plsc.subcore_barrier()

    @pl.loop(0, NMAIN, step=NBUF)
    def _(k0):
        for b in range(NBUF):
            k = k0 + b
            b1 = (b + 1) % NBUF

            @pl.when(k >= 3)
            def _():
                wait_scatter(b1, b1)

            wait_idx(k + 1, b1)
            issue_gather(b1, b1)

            @pl.when(k + 2 < NCHUNK)
            def _():
                issue_idx(k + 2, (b + 2) % NBUF)

            wait_gather(b, b)
            scale_chunk(b, b)
            issue_scatter(b, b)

    tb = (NCHUNK - 1) % NBUF
    wait_gather(tb, tb)
    scale_chunk(tb, tb)
    issue_scatter(tb, tb)
    for k in range(NCHUNK - NBUF, NCHUNK):
        wait_scatter(k % NBUF, k % NBUF)

    plsc.subcore_barrier()

    @pl.when(c == 0)
    def _():
        pltpu.sync_copy(agg_sh.at[pl.ds(s * RPS, RPS)],
                        out0_hbm.at[pl.ds(s * RPS, RPS)])

    @pl.when(c == 1)
    def _():
        pltpu.sync_copy(agg_sh.at[pl.ds(s * RPS, RPS)],
                        out1_hbm.at[pl.ds(s * RPS, RPS)])


def _sc_agg(x, src, dst, w):
    return pl.kernel(
        _sc_agg_body,
        out_type=(jax.ShapeDtypeStruct((NP, D), jnp.float32),
                  jax.ShapeDtypeStruct((NP, D), jnp.float32)),
        mesh=plsc.VectorSubcoreMesh(core_axis_name="c", subcore_axis_name="s",
                                    num_cores=NC, num_subcores=NS),
        scratch_types=[
            pltpu.VMEM_SHARED((NP, D), jnp.float32),
            tuple(pltpu.VMEM((CH, D), jnp.float32) for _ in range(NBUF)),
            pltpu.VMEM((NBUF, CH), jnp.int32),
            pltpu.VMEM((NBUF, CH), jnp.int32),
            pltpu.VMEM((NBUF, CH), jnp.float32),
            tuple(pltpu.SemaphoreType.DMA for _ in range(NBUF)),
            tuple(pltpu.SemaphoreType.DMA for _ in range(NBUF)),
            tuple(pltpu.SemaphoreType.DMA for _ in range(NBUF)),
        ],
    )(x, src, dst, w)


RB = 400
NRB = N // RB


def _pre_body(batch_ref, x_ref, wseq_ref, wroot_ref, bseq_ref, bstruc_ref,
              pre_ref, sums_ref, cnts_ref, mean_ref):
    p = pl.program_id(0)
    i = pl.program_id(1)
    b = batch_ref[0]
    oh_t = (lax.broadcasted_iota(jnp.int32, (G, RB), 0) == b).astype(jnp.float32)

    @pl.when(p == 0)
    def _():
        @pl.when(i == 0)
        def _():
            sums_ref[...] = jnp.zeros_like(sums_ref)
            cnts_ref[...] = jnp.zeros_like(cnts_ref)

        sums_ref[...] += jnp.dot(oh_t, x_ref[...],
                                 preferred_element_type=jnp.float32)
        cnts_ref[...] += jnp.dot(oh_t, jnp.ones((RB, D), jnp.float32),
                                 preferred_element_type=jnp.float32)

        @pl.when(i == NRB - 1)
        def _():
            mean_ref[...] = sums_ref[...] / jnp.maximum(cnts_ref[...], 1.0)

    @pl.when(p == 1)
    def _():
        ctx = lax.dot_general(oh_t, mean_ref[...],
                              dimension_numbers=(((0,), (0,)), ((), ())),
                              preferred_element_type=jnp.float32)
        acc = jnp.dot(x_ref[...], wseq_ref[...] + wroot_ref[...],
                      preferred_element_type=jnp.float32)
        acc += jnp.dot(ctx, wseq_ref[...], preferred_element_type=jnp.float32)
        pre_ref[...] = acc + bseq_ref[...] + bstruc_ref[...]


def _tail_body(pre_ref, agg0_ref, agg1_ref, wnbr_ref, o_ref):
    agg = agg0_ref[...] + agg1_ref[...]
    acc = pre_ref[...] + jnp.dot(agg, wnbr_ref[...],
                                 preferred_element_type=jnp.float32)
    o_ref[...] = jnp.maximum(acc, 0.0)


def kernel(x, edge_index, edge_weight, batch, W_seq, b_seq, W_root, W_nbr, b_struc):
    src = edge_index[0]
    dst = edge_index[1]
    agg0, agg1 = _sc_agg(x, src, dst, edge_weight)

    batch3 = batch.reshape(NRB, 1, RB)

    pre = pl.pallas_call(
        _pre_body,
        grid=(2, NRB),
        in_specs=[
            pl.BlockSpec((1, 1, RB), lambda p, i: (i, 0, 0)),
            pl.BlockSpec((RB, D), lambda p, i: (i, 0)),
            pl.BlockSpec((D, D), lambda p, i: (0, 0)),
            pl.BlockSpec((D, D), lambda p, i: (0, 0)),
            pl.BlockSpec((1, D), lambda p, i: (0, 0)),
            pl.BlockSpec((1, D), lambda p, i: (0, 0)),
        ],
        out_specs=pl.BlockSpec((RB, D), lambda p, i: (i, 0)),
        out_shape=jax.ShapeDtypeStruct((N, D), jnp.float32),
        scratch_shapes=[pltpu.VMEM((G, D), jnp.float32),
                        pltpu.VMEM((G, D), jnp.float32),
                        pltpu.VMEM((G, D), jnp.float32)],
    )(batch3, x, W_seq, W_root, b_seq.reshape(1, D), b_struc.reshape(1, D))

    out = pl.pallas_call(
        _tail_body,
        grid=(NRB,),
        in_specs=[
            pl.BlockSpec((RB, D), lambda i: (i, 0)),
            pl.BlockSpec((RB, D), lambda i: (i, 0)),
            pl.BlockSpec((RB, D), lambda i: (i, 0)),
            pl.BlockSpec((D, D), lambda i: (0, 0)),
        ],
        out_specs=pl.BlockSpec((RB, D), lambda i: (i, 0)),
        out_shape=jax.ShapeDtypeStruct((N, D), jnp.float32),
    )(pre, agg0, agg1, W_nbr)
    return out


# speedup vs baseline: 1.1900x; 1.1900x over previous
import functools

import jax
import jax.numpy as jnp
from jax import lax
from jax.experimental import pallas as pl
from jax.experimental.pallas import tpu as pltpu
from jax.experimental.pallas import tpu_sc as plsc

N, E, D, G = 10000, 320000, 128, 256
NC, NS = 2, 16
NW = NC * NS
EPW = E // NW
CH = 80
NCHUNK = EPW // CH
NP = 10240
RPS = NP // NS
ZR = 128
NLANE = 16
NSL = D // NLANE


NBUF = 4
NMAIN = (NCHUNK // NBUF) * NBUF


def _sc_agg_body(x_hbm, src_hbm, dst_hbm, w_hbm, out0_hbm, out1_hbm,
                 agg_sh, rows, src_r, dst_r, w_r, gsems, ssems, isems):
    c = lax.axis_index("c")
    s = lax.axis_index("s")
    wid = c * NS + s

    base = wid * EPW

    def zrow(i, carry):
        for sl in range(NSL):
            rows[NBUF - 1][i, pl.ds(sl * NLANE, NLANE)] = jnp.zeros(
                (NLANE,), jnp.float32)
        return carry

    lax.fori_loop(0, CH, zrow, 0)
    for i in range(RPS // CH):
        pltpu.async_copy(rows[NBUF - 1],
                         agg_sh.at[pl.ds(s * RPS + i * CH, CH)], ssems[0])

    def issue_idx(k, r):
        e0 = base + k * CH
        pltpu.async_copy(src_hbm.at[pl.ds(e0, CH)], src_r.at[r], isems[r])
        pltpu.async_copy(dst_hbm.at[pl.ds(e0, CH)], dst_r.at[r], isems[r])
        pltpu.async_copy(w_hbm.at[pl.ds(e0, CH)], w_r.at[r], isems[r])

    def wait_idx(k, r):
        e0 = base + k * CH
        pltpu.make_async_copy(src_hbm.at[pl.ds(e0, CH)], src_r.at[r], isems[r]).wait()
        pltpu.make_async_copy(dst_hbm.at[pl.ds(e0, CH)], dst_r.at[r], isems[r]).wait()
        pltpu.make_async_copy(w_hbm.at[pl.ds(e0, CH)], w_r.at[r], isems[r]).wait()

    def issue_gather(r, b):
        pltpu.async_copy(x_hbm.at[src_r.at[r]], rows[b], gsems[b])

    def wait_gather(r, b):
        pltpu.make_async_copy(x_hbm.at[src_r.at[r]], rows[b], gsems[b]).wait()

    def issue_scatter(r, b):
        pltpu.async_copy(rows[b], agg_sh.at[dst_r.at[r]], ssems[b], add=True)

    def wait_scatter(r, b):
        pltpu.make_async_copy(rows[b], agg_sh.at[dst_r.at[r]], ssems[b]).wait()

    def scale_chunk(r, b):
        def scale(g, inner):
            wvec = w_r[r, pl.ds(g * NLANE, NLANE)]
            for j in range(NLANE):
                e = g * NLANE + j
                w = wvec[j]
                for sl in range(NSL):
                    rows[b][e, pl.ds(sl * NLANE, NLANE)] = (
                        rows[b][e, pl.ds(sl * NLANE, NLANE)] * w)
            return inner

        lax.fori_loop(0, CH // NLANE, scale, 0)

    issue_idx(0, 0)
    issue_idx(1, 1)
    wait_idx(0, 0)
    issue_gather(0, 0)
    for i in range(RPS // CH):
        pltpu.make_async_copy(rows[NBUF - 1],
                              agg_sh.at[pl.ds(s * RPS + i * CH, CH)],
                              ssems[0]).wait()
    plsc.subcore_barrier()

    @pl.loop(0, NMAIN, step=NBUF)
    def _(k0):
        for b in range(NBUF):
            k = k0 + b
            b1 = (b + 1) % NBUF

            @pl.when(jnp.logical_and(k >= NBUF - 1, k + 1 < NCHUNK))
            def _():
                wait_scatter(b1, b1)

            @pl.when(k + 1 < NCHUNK)
            def _():
                wait_idx(k + 1, b1)
                issue_gather(b1, b1)

            @pl.when(k + 2 < NCHUNK)
            def _():
                issue_idx(k + 2, (b + 2) % NBUF)

            wait_gather(b, b)
            scale_chunk(b, b)
            issue_scatter(b, b)

    if NMAIN < NCHUNK:
        tb = (NCHUNK - 1) % NBUF
        wait_gather(tb, tb)
        scale_chunk(tb, tb)
        issue_scatter(tb, tb)
    for k in range(NCHUNK - NBUF, NCHUNK):
        wait_scatter(k % NBUF, k % NBUF)

    plsc.subcore_barrier()

    @pl.when(c == 0)
    def _():
        pltpu.sync_copy(agg_sh.at[pl.ds(s * RPS, RPS)],
                        out0_hbm.at[pl.ds(s * RPS, RPS)])

    @pl.when(c == 1)
    def _():
        pltpu.sync_copy(agg_sh.at[pl.ds(s * RPS, RPS)],
                        out1_hbm.at[pl.ds(s * RPS, RPS)])


def _sc_agg(x, src, dst, w):
    return pl.kernel(
        _sc_agg_body,
        out_type=(jax.ShapeDtypeStruct((NP, D), jnp.float32),
                  jax.ShapeDtypeStruct((NP, D), jnp.float32)),
        mesh=plsc.VectorSubcoreMesh(core_axis_name="c", subcore_axis_name="s",
                                    num_cores=NC, num_subcores=NS),
        scratch_types=[
            pltpu.VMEM_SHARED((NP, D), jnp.float32),
            tuple(pltpu.VMEM((CH, D), jnp.float32) for _ in range(NBUF)),
            pltpu.VMEM((NBUF, CH), jnp.int32),
            pltpu.VMEM((NBUF, CH), jnp.int32),
            pltpu.VMEM((NBUF, CH), jnp.float32),
            tuple(pltpu.SemaphoreType.DMA for _ in range(NBUF)),
            tuple(pltpu.SemaphoreType.DMA for _ in range(NBUF)),
            tuple(pltpu.SemaphoreType.DMA for _ in range(NBUF)),
        ],
    )(x, src, dst, w)


RB = 400
NRB = N // RB


def _pre_body(batch_ref, x_ref, wseq_ref, wroot_ref, bseq_ref, bstruc_ref,
              pre_ref, sums_ref, cnts_ref, mean_ref):
    p = pl.program_id(0)
    i = pl.program_id(1)
    b = batch_ref[0]
    oh_t = (lax.broadcasted_iota(jnp.int32, (G, RB), 0) == b).astype(jnp.float32)

    @pl.when(p == 0)
    def _():
        @pl.when(i == 0)
        def _():
            sums_ref[...] = jnp.zeros_like(sums_ref)
            cnts_ref[...] = jnp.zeros_like(cnts_ref)

        sums_ref[...] += jnp.dot(oh_t, x_ref[...],
                                 preferred_element_type=jnp.float32)
        cnts_ref[...] += jnp.dot(oh_t, jnp.ones((RB, D), jnp.float32),
                                 preferred_element_type=jnp.float32)

        @pl.when(i == NRB - 1)
        def _():
            mean_ref[...] = sums_ref[...] / jnp.maximum(cnts_ref[...], 1.0)

    @pl.when(p == 1)
    def _():
        ctx = lax.dot_general(oh_t, mean_ref[...],
                              dimension_numbers=(((0,), (0,)), ((), ())),
                              preferred_element_type=jnp.float32)
        acc = jnp.dot(x_ref[...], wseq_ref[...] + wroot_ref[...],
                      preferred_element_type=jnp.float32)
        acc += jnp.dot(ctx, wseq_ref[...], preferred_element_type=jnp.float32)
        pre_ref[...] = acc + bseq_ref[...] + bstruc_ref[...]


def _tail_body(pre_ref, agg0_ref, agg1_ref, wnbr_ref, o_ref):
    agg = agg0_ref[...] + agg1_ref[...]
    acc = pre_ref[...] + jnp.dot(agg, wnbr_ref[...],
                                 preferred_element_type=jnp.float32)
    o_ref[...] = jnp.maximum(acc, 0.0)


def kernel(x, edge_index, edge_weight, batch, W_seq, b_seq, W_root, W_nbr, b_struc):
    src = edge_index[0]
    dst = edge_index[1]
    agg0, agg1 = _sc_agg(x, src, dst, edge_weight)

    batch3 = batch.reshape(NRB, 1, RB)

    pre = pl.pallas_call(
        _pre_body,
        grid=(2, NRB),
        in_specs=[
            pl.BlockSpec((1, 1, RB), lambda p, i: (i, 0, 0)),
            pl.BlockSpec((RB, D), lambda p, i: (i, 0)),
            pl.BlockSpec((D, D), lambda p, i: (0, 0)),
            pl.BlockSpec((D, D), lambda p, i: (0, 0)),
            pl.BlockSpec((1, D), lambda p, i: (0, 0)),
            pl.BlockSpec((1, D), lambda p, i: (0, 0)),
        ],
        out_specs=pl.BlockSpec((RB, D), lambda p, i: (i, 0)),
        out_shape=jax.ShapeDtypeStruct((N, D), jnp.float32),
        scratch_shapes=[pltpu.VMEM((G, D), jnp.float32),
                        pltpu.VMEM((G, D), jnp.float32),
                        pltpu.VMEM((G, D), jnp.float32)],
    )(batch3, x, W_seq, W_root, b_seq.reshape(1, D), b_struc.reshape(1, D))

    out = pl.pallas_call(
        _tail_body,
        grid=(NRB,),
        in_specs=[
            pl.BlockSpec((RB, D), lambda i: (i, 0)),
            pl.BlockSpec((RB, D), lambda i: (i, 0)),
            pl.BlockSpec((RB, D), lambda i: (i, 0)),
            pl.BlockSpec((D, D), lambda i: (0, 0)),
        ],
        out_specs=pl.BlockSpec((RB, D), lambda i: (i, 0)),
        out_shape=jax.ShapeDtypeStruct((N, D), jnp.float32),
    )(pre, agg0, agg1, W_nbr)
    return out
